# baseline (speedup 1.0000x reference)
import functools

import jax
import jax.numpy as jnp
from jax import lax
from jax.experimental import pallas as pl
from jax.experimental.pallas import tpu as pltpu
from jax.experimental.pallas import tpu_sc as plsc

NC = 2
NS = 16
NW = NC * NS
LANES = 16
CHUNK = 32
NBUF = 5
LEAD = 3
ROW_BLK = 1024


def _ceil_to(v, m):
  return (v + m - 1) // m * m


def _make_deg_kernel(spw, rt):
  mesh = plsc.VectorSubcoreMesh(core_axis_name="c", subcore_axis_name="s")
  rpt = rt // NS

  @functools.partial(
      pl.kernel,
      mesh=mesh,
      out_type=jax.ShapeDtypeStruct((NC * rt,), jnp.float32),
      scratch_types=[
          pltpu.VMEM((spw,), jnp.int32),
          pltpu.VMEM((rt,), jnp.float32),
          pltpu.VMEM((NS * rpt,), jnp.float32),
          pltpu.VMEM((rpt,), jnp.float32),
          pltpu.VMEM_SHARED((NS * rt,), jnp.float32),
      ],
      compiler_params=pltpu.CompilerParams(needs_layout_passes=False),
  )
  def deg_kernel(dst_hbm, out_hbm, idx_v, tbl_v, stage_v, red_v, tbls_s):
    cid = lax.axis_index("c")
    sid = lax.axis_index("s")
    wid = sid * NC + cid
    pltpu.sync_copy(dst_hbm.at[pl.ds(wid * spw, spw)], idx_v)

    zeros16 = jnp.zeros((LANES,), jnp.float32)
    ones16 = jnp.ones((LANES,), jnp.float32)

    def zero_body(i, _):
      tbl_v[pl.ds(i * LANES, LANES)] = zeros16
      return 0

    lax.fori_loop(0, rt // LANES, zero_body, 0)

    def count_body(i, _):
      idx = idx_v[pl.ds(i * LANES, LANES)]
      plsc.addupdate_scatter(tbl_v, [idx], ones16)
      return 0

    lax.fori_loop(0, spw // LANES, count_body, 0)
    pltpu.sync_copy(tbl_v, tbls_s.at[pl.ds(sid * rt, rt)])
    plsc.subcore_barrier()

    for k in range(NS):
      pltpu.sync_copy(tbls_s.at[pl.ds(k * rt + sid * rpt, rpt)],
                      stage_v.at[pl.ds(k * rpt, rpt)])

    def red_body(i, _):
      acc = zeros16
      for k in range(NS):
        acc = acc + stage_v[pl.ds(k * rpt + i * LANES, LANES)]
      red_v[pl.ds(i * LANES, LANES)] = acc
      return 0

    lax.fori_loop(0, rpt // LANES, red_body, 0)
    pltpu.sync_copy(red_v, out_hbm.at[pl.ds(cid * rt + sid * rpt, rpt)])

  return deg_kernel


def _make_agg_kernel(cpw, rt, dh):
  mesh = plsc.VectorSubcoreMesh(core_axis_name="c", subcore_axis_name="s")
  rpt = rt // NS
  assert cpw % NBUF == 0 and cpw > NBUF

  @functools.partial(
      pl.kernel,
      mesh=mesh,
      out_type=jax.ShapeDtypeStruct((NC * rt, dh), jnp.float32),
      scratch_types=[
          pltpu.VMEM((cpw * CHUNK,), jnp.int32),
          pltpu.VMEM((cpw, CHUNK), jnp.int32),
          pltpu.VMEM((NBUF, CHUNK, dh), jnp.float32),
          pltpu.VMEM_SHARED((rt, dh), jnp.float32),
      ] + [pltpu.SemaphoreType.DMA] * (2 * NBUF),
      compiler_params=pltpu.CompilerParams(needs_layout_passes=False),
  )
  def agg_kernel(hs_hbm, src_hbm, dst_hbm, zeros_hbm, out_hbm,
                 src_v, dst_v, rows_v, acc_s, *sems):
    gsem = sems[:NBUF]
    ssem = sems[NBUF:]
    cid = lax.axis_index("c")
    sid = lax.axis_index("s")
    wid = sid * NC + cid

    pltpu.sync_copy(zeros_hbm, acc_s.at[pl.ds(sid * rpt, rpt)])
    pltpu.sync_copy(src_hbm.at[pl.ds(wid * cpw * CHUNK, cpw * CHUNK)], src_v)
    pltpu.sync_copy(dst_hbm.at[wid], dst_v)
    plsc.subcore_barrier()

    def gather(j, b):
      pltpu.async_copy(hs_hbm.at[src_v.at[pl.ds(j * CHUNK, CHUNK)]],
                       rows_v.at[b], gsem[b])

    def scatter(j, b):
      pltpu.async_copy(rows_v.at[b], acc_s.at[dst_v.at[j]], ssem[b],
                       add=True)

    for b in range(LEAD):
      gather(b, b)

    def round_body(g, _):
      for b in range(NBUF):
        j = g * NBUF + b
        pltpu.make_async_copy(
            hs_hbm.at[src_v.at[pl.ds(j * CHUNK, CHUNK)]],
            rows_v.at[b], gsem[b]).wait()
        scatter(j, b)
        jn = j + LEAD
        bn = (b + LEAD) % NBUF

        @pl.when(jn < cpw)
        def _():
          @pl.when(jn >= NBUF)
          def _():
            pltpu.make_async_copy(
                rows_v.at[bn], acc_s.at[dst_v.at[j]], ssem[bn]).wait()
          gather(jn, bn)

      return 0

    lax.fori_loop(0, cpw // NBUF, round_body, 0)

    for b in range(NBUF):
      pltpu.make_async_copy(rows_v.at[b], acc_s.at[dst_v.at[0]],
                            ssem[b]).wait()
    plsc.subcore_barrier()

    pltpu.sync_copy(
        acc_s.at[pl.ds(sid * rpt, rpt)],
        out_hbm.at[pl.ds(cid * rt + sid * rpt, rpt)],
    )

  return agg_kernel


def _split_body(ei_ref, src_ref, dst_ref):
  src_ref[...] = ei_ref[0, :]
  dst_ref[...] = ei_ref[1, :]


def _mm_body(x_ref, w_ref, h_ref):
  h_ref[...] = jnp.dot(x_ref[...].astype(jnp.bfloat16),
                       w_ref[...].astype(jnp.bfloat16),
                       preferred_element_type=jnp.float32)


def _dinv_from(degp_ref, i):
  rt = degp_ref.shape[0] // NC
  deg = 1.0 + degp_ref[pl.ds(i * ROW_BLK, ROW_BLK)]
  for c in range(1, NC):
    deg = deg + degp_ref[pl.ds(c * rt + i * ROW_BLK, ROW_BLK)]
  return lax.rsqrt(deg)


def _scale_body(h_ref, degp_ref, hs_ref):
  dinv = _dinv_from(degp_ref, pl.program_id(0))
  hs_ref[...] = h_ref[...] * dinv[:, None]


def _leaky(v):
  return jnp.where(v >= 0.0, v, 0.01 * v)


def _tail_body(p0_ref, p1_ref, hs_ref, degp_ref, bg_ref, w1_ref, b1_ref,
               w3_ref, b3_ref, out_ref):
  dinv = _dinv_from(degp_ref, pl.program_id(0))
  agg = p0_ref[...] + p1_ref[...] + hs_ref[...]
  g1 = _leaky(agg * dinv[:, None] + bg_ref[...][None, :])
  l1 = _leaky(
      jnp.dot(g1, w1_ref[...], preferred_element_type=jnp.float32)
      + b1_ref[...][None, :])
  res = (
      jnp.dot(l1, w3_ref[...], preferred_element_type=jnp.float32)
      + b3_ref[...][None, :])
  out_ref[...] = res.T


def kernel(x, edge_index, batch, W_gcn, b_gcn, W1, b1, W3, b3):
  n, din = x.shape
  dh = W_gcn.shape[1]
  lat = W1.shape[1]
  e = edge_index.shape[1]

  rt = _ceil_to(n + LANES, 512)
  cpw = _ceil_to(-(-e // (NW * CHUNK)), NBUF)
  ep = NW * CHUNK * cpw
  spw = cpw * CHUNK

  src_e, dst_e = pl.pallas_call(
      _split_body,
      out_shape=[
          jax.ShapeDtypeStruct((e,), jnp.int32),
          jax.ShapeDtypeStruct((e,), jnp.int32),
      ],
  )(edge_index)

  pad_idx = jnp.arange(ep - e, dtype=jnp.int32)
  src = jnp.concatenate([src_e, pad_idx % n])
  dst = jnp.concatenate([dst_e, n + pad_idx % (rt - n)])
  dst3 = dst.reshape(NW, cpw, CHUNK)

  zeros_tile = jnp.zeros((rt // NS, dh), jnp.float32)

  deg_partials = _make_deg_kernel(spw, rt)(dst)

  grid = (-(-n // ROW_BLK),)
  h = pl.pallas_call(
      _mm_body,
      grid=grid,
      in_specs=[
          pl.BlockSpec((ROW_BLK, din), lambda i: (i, 0)),
          pl.BlockSpec((din, dh), lambda i: (0, 0)),
      ],
      out_specs=pl.BlockSpec((ROW_BLK, dh), lambda i: (i, 0)),
      out_shape=jax.ShapeDtypeStruct((n, dh), jnp.float32),
  )(x, W_gcn)

  hs = pl.pallas_call(
      _scale_body,
      grid=grid,
      in_specs=[
          pl.BlockSpec((ROW_BLK, dh), lambda i: (i, 0)),
          pl.BlockSpec((NC * rt,), lambda i: (0,)),
      ],
      out_specs=pl.BlockSpec((ROW_BLK, dh), lambda i: (i, 0)),
      out_shape=jax.ShapeDtypeStruct((n, dh), jnp.float32),
  )(h, deg_partials)

  partials = _make_agg_kernel(cpw, rt, dh)(hs, src, dst3, zeros_tile)

  nb = rt // ROW_BLK
  out_t = pl.pallas_call(
      _tail_body,
      grid=grid,
      in_specs=[
          pl.BlockSpec((ROW_BLK, dh), lambda i: (i, 0)),
          pl.BlockSpec((ROW_BLK, dh), lambda i: (nb + i, 0)),
          pl.BlockSpec((ROW_BLK, dh), lambda i: (i, 0)),
          pl.BlockSpec((NC * rt,), lambda i: (0,)),
          pl.BlockSpec((dh,), lambda i: (0,)),
          pl.BlockSpec((dh, lat), lambda i: (0, 0)),
          pl.BlockSpec((lat,), lambda i: (0,)),
          pl.BlockSpec((lat, lat), lambda i: (0, 0)),
          pl.BlockSpec((lat,), lambda i: (0,)),
      ],
      out_specs=pl.BlockSpec((lat, ROW_BLK), lambda i: (0, i)),
      out_shape=jax.ShapeDtypeStruct((lat, n), jnp.float32),
  )(partials, partials, hs, deg_partials, b_gcn, W1, b1, W3, b3)

  return out_t.T


# speedup vs baseline: 1.0511x; 1.0511x over previous
import functools

import jax
import jax.numpy as jnp
from jax import lax
from jax.experimental import pallas as pl
from jax.experimental.pallas import tpu as pltpu
from jax.experimental.pallas import tpu_sc as plsc

NC = 2
NS = 16
NW = NC * NS
LANES = 16
CHUNK = 64
NBUF = 3
LEAD = 2
ROW_BLK = 1024


def _ceil_to(v, m):
  return (v + m - 1) // m * m


def _make_deg_kernel(spw, rt):
  mesh = plsc.VectorSubcoreMesh(core_axis_name="c", subcore_axis_name="s")
  rpt = rt // NS

  @functools.partial(
      pl.kernel,
      mesh=mesh,
      out_type=jax.ShapeDtypeStruct((NC * rt,), jnp.float32),
      scratch_types=[
          pltpu.VMEM((spw,), jnp.int32),
          pltpu.VMEM((rt,), jnp.float32),
          pltpu.VMEM((NS * rpt,), jnp.float32),
          pltpu.VMEM((rpt,), jnp.float32),
          pltpu.VMEM_SHARED((NS * rt,), jnp.float32),
      ],
      compiler_params=pltpu.CompilerParams(needs_layout_passes=False),
  )
  def deg_kernel(dst_hbm, out_hbm, idx_v, tbl_v, stage_v, red_v, tbls_s):
    cid = lax.axis_index("c")
    sid = lax.axis_index("s")
    wid = sid * NC + cid
    pltpu.sync_copy(dst_hbm.at[pl.ds(wid * spw, spw)], idx_v)

    zeros16 = jnp.zeros((LANES,), jnp.float32)
    ones16 = jnp.ones((LANES,), jnp.float32)

    def zero_body(i, _):
      tbl_v[pl.ds(i * LANES, LANES)] = zeros16
      return 0

    lax.fori_loop(0, rt // LANES, zero_body, 0)

    def count_body(i, _):
      idx = idx_v[pl.ds(i * LANES, LANES)]
      plsc.addupdate_scatter(tbl_v, [idx], ones16)
      return 0

    lax.fori_loop(0, spw // LANES, count_body, 0)
    pltpu.sync_copy(tbl_v, tbls_s.at[pl.ds(sid * rt, rt)])
    plsc.subcore_barrier()

    for k in range(NS):
      pltpu.sync_copy(tbls_s.at[pl.ds(k * rt + sid * rpt, rpt)],
                      stage_v.at[pl.ds(k * rpt, rpt)])

    def red_body(i, _):
      acc = zeros16
      for k in range(NS):
        acc = acc + stage_v[pl.ds(k * rpt + i * LANES, LANES)]
      red_v[pl.ds(i * LANES, LANES)] = acc
      return 0

    lax.fori_loop(0, rpt // LANES, red_body, 0)
    pltpu.sync_copy(red_v, out_hbm.at[pl.ds(cid * rt + sid * rpt, rpt)])

  return deg_kernel


def _make_agg_kernel(cpw, rt, dh):
  mesh = plsc.VectorSubcoreMesh(core_axis_name="c", subcore_axis_name="s")
  rpt = rt // NS
  assert cpw % NBUF == 0 and cpw > NBUF

  @functools.partial(
      pl.kernel,
      mesh=mesh,
      out_type=jax.ShapeDtypeStruct((NC * rt, dh), jnp.float32),
      scratch_types=[
          pltpu.VMEM((cpw * CHUNK,), jnp.int32),
          pltpu.VMEM((cpw, CHUNK), jnp.int32),
          pltpu.VMEM((NBUF, CHUNK, dh), jnp.float32),
          pltpu.VMEM_SHARED((rt, dh), jnp.float32),
          pltpu.SemaphoreType.DMA,
          pltpu.SemaphoreType.DMA,
          pltpu.SemaphoreType.DMA,
          pltpu.SemaphoreType.DMA,
          pltpu.SemaphoreType.DMA,
          pltpu.SemaphoreType.DMA,
      ],
      compiler_params=pltpu.CompilerParams(needs_layout_passes=False),
  )
  def agg_kernel(hs_hbm, src_hbm, dst_hbm, zeros_hbm, out_hbm,
                 src_v, dst_v, rows_v, acc_s, *sems):
    gsem = sems[:NBUF]
    ssem = sems[NBUF:]
    cid = lax.axis_index("c")
    sid = lax.axis_index("s")
    wid = sid * NC + cid

    pltpu.sync_copy(zeros_hbm, acc_s.at[pl.ds(sid * rpt, rpt)])
    pltpu.sync_copy(src_hbm.at[pl.ds(wid * cpw * CHUNK, cpw * CHUNK)], src_v)
    pltpu.sync_copy(dst_hbm.at[wid], dst_v)
    plsc.subcore_barrier()

    def gather(j, b):
      pltpu.async_copy(hs_hbm.at[src_v.at[pl.ds(j * CHUNK, CHUNK)]],
                       rows_v.at[b], gsem[b])

    def scatter(j, b):
      pltpu.async_copy(rows_v.at[b], acc_s.at[dst_v.at[j]], ssem[b],
                       add=True)

    for b in range(LEAD):
      gather(b, b)

    def round_body(g, _):
      for b in range(NBUF):
        j = g * NBUF + b
        pltpu.make_async_copy(
            hs_hbm.at[src_v.at[pl.ds(j * CHUNK, CHUNK)]],
            rows_v.at[b], gsem[b]).wait()
        scatter(j, b)
        jn = j + LEAD
        bn = (b + LEAD) % NBUF

        @pl.when(jn < cpw)
        def _():
          @pl.when(jn >= NBUF)
          def _():
            pltpu.make_async_copy(
                rows_v.at[bn], acc_s.at[dst_v.at[j]], ssem[bn]).wait()
          gather(jn, bn)

      return 0

    lax.fori_loop(0, cpw // NBUF, round_body, 0)

    for b in range(NBUF):
      pltpu.make_async_copy(rows_v.at[b], acc_s.at[dst_v.at[0]],
                            ssem[b]).wait()
    plsc.subcore_barrier()

    pltpu.sync_copy(
        acc_s.at[pl.ds(sid * rpt, rpt)],
        out_hbm.at[pl.ds(cid * rt + sid * rpt, rpt)],
    )

  return agg_kernel


def _split_body(ei_ref, src_ref, dst_ref):
  src_ref[...] = ei_ref[0, :]
  dst_ref[...] = ei_ref[1, :]


def _mm_body(x_ref, w_ref, h_ref):
  h_ref[...] = jnp.dot(x_ref[...].astype(jnp.bfloat16),
                       w_ref[...].astype(jnp.bfloat16),
                       preferred_element_type=jnp.float32)


def _dinv_from(degp_ref, i):
  rt = degp_ref.shape[0] // NC
  deg = 1.0 + degp_ref[pl.ds(i * ROW_BLK, ROW_BLK)]
  for c in range(1, NC):
    deg = deg + degp_ref[pl.ds(c * rt + i * ROW_BLK, ROW_BLK)]
  return lax.rsqrt(deg)


def _scale_body(h_ref, degp_ref, hs_ref):
  dinv = _dinv_from(degp_ref, pl.program_id(0))
  hs_ref[...] = h_ref[...] * dinv[:, None]


def _leaky(v):
  return jnp.where(v >= 0.0, v, 0.01 * v)


def _tail_body(p0_ref, p1_ref, hs_ref, degp_ref, bg_ref, w1_ref, b1_ref,
               w3_ref, b3_ref, out_ref):
  dinv = _dinv_from(degp_ref, pl.program_id(0))
  agg = p0_ref[...] + p1_ref[...] + hs_ref[...]
  g1 = _leaky(agg * dinv[:, None] + bg_ref[...][None, :])
  l1 = _leaky(
      jnp.dot(g1, w1_ref[...], preferred_element_type=jnp.float32)
      + b1_ref[...][None, :])
  res = (
      jnp.dot(l1, w3_ref[...], preferred_element_type=jnp.float32)
      + b3_ref[...][None, :])
  out_ref[...] = res.T


def kernel(x, edge_index, batch, W_gcn, b_gcn, W1, b1, W3, b3):
  n, din = x.shape
  dh = W_gcn.shape[1]
  lat = W1.shape[1]
  e = edge_index.shape[1]

  rt = _ceil_to(n + LANES, 512)
  cpw = _ceil_to(-(-e // (NW * CHUNK)), NBUF)
  ep = NW * CHUNK * cpw
  spw = cpw * CHUNK

  src_e, dst_e = pl.pallas_call(
      _split_body,
      out_shape=[
          jax.ShapeDtypeStruct((e,), jnp.int32),
          jax.ShapeDtypeStruct((e,), jnp.int32),
      ],
  )(edge_index)

  pad_idx = jnp.arange(ep - e, dtype=jnp.int32)
  src = jnp.concatenate([src_e, pad_idx % n])
  dst = jnp.concatenate([dst_e, n + pad_idx % (rt - n)])
  dst3 = dst.reshape(NW, cpw, CHUNK)

  zeros_tile = jnp.zeros((rt // NS, dh), jnp.float32)

  deg_partials = _make_deg_kernel(spw, rt)(dst)

  grid = (-(-n // ROW_BLK),)
  h = pl.pallas_call(
      _mm_body,
      grid=grid,
      in_specs=[
          pl.BlockSpec((ROW_BLK, din), lambda i: (i, 0)),
          pl.BlockSpec((din, dh), lambda i: (0, 0)),
      ],
      out_specs=pl.BlockSpec((ROW_BLK, dh), lambda i: (i, 0)),
      out_shape=jax.ShapeDtypeStruct((n, dh), jnp.float32),
  )(x, W_gcn)

  hs = pl.pallas_call(
      _scale_body,
      grid=grid,
      in_specs=[
          pl.BlockSpec((ROW_BLK, dh), lambda i: (i, 0)),
          pl.BlockSpec((NC * rt,), lambda i: (0,)),
      ],
      out_specs=pl.BlockSpec((ROW_BLK, dh), lambda i: (i, 0)),
      out_shape=jax.ShapeDtypeStruct((n, dh), jnp.float32),
  )(h, deg_partials)

  partials = _make_agg_kernel(cpw, rt, dh)(hs, src, dst3, zeros_tile)

  nb = rt // ROW_BLK
  out_t = pl.pallas_call(
      _tail_body,
      grid=grid,
      in_specs=[
          pl.BlockSpec((ROW_BLK, dh), lambda i: (i, 0)),
          pl.BlockSpec((ROW_BLK, dh), lambda i: (nb + i, 0)),
          pl.BlockSpec((ROW_BLK, dh), lambda i: (i, 0)),
          pl.BlockSpec((NC * rt,), lambda i: (0,)),
          pl.BlockSpec((dh,), lambda i: (0,)),
          pl.BlockSpec((dh, lat), lambda i: (0, 0)),
          pl.BlockSpec((lat,), lambda i: (0,)),
          pl.BlockSpec((lat, lat), lambda i: (0, 0)),
          pl.BlockSpec((lat,), lambda i: (0,)),
      ],
      out_specs=pl.BlockSpec((lat, ROW_BLK), lambda i: (0, i)),
      out_shape=jax.ShapeDtypeStruct((lat, n), jnp.float32),
  )(partials, partials, hs, deg_partials, b_gcn, W1, b1, W3, b3)

  return out_t.T
